# baseline (speedup 1.0000x reference)
import functools

import jax
import jax.numpy as jnp
from jax import lax
from jax.experimental import pallas as pl
from jax.experimental.pallas import tpu as pltpu
from jax.experimental.pallas import tpu_sc as plsc

_B, _G, _C, _K = 4, 8192, 128, 3
_FM = (128, 128)
_GX, _GY = _FM[0] + 1, _FM[1] + 1
_N = _B * _G
_BM = 512
_NT = _N + _BM
_NO = _K * _K
_NC, _NS = 2, 16
_NW = _NC * _NS
_PW = _N // _NW
_P = 16
_CH = _PW // _P


_mm_in_dtype = jnp.bfloat16


def _mm_body(f_ref, w_ref, z_ref):
    f = f_ref[...]
    for o in range(_NO):
        z_ref[o] = jnp.dot(f, w_ref[o], preferred_element_type=jnp.float32)


_mm = pl.pallas_call(
    _mm_body,
    grid=(_NT // _BM,),
    in_specs=[
        pl.BlockSpec((_BM, _C), lambda i: (i, 0)),
        pl.BlockSpec((_NO, _C, _C), lambda i: (0, 0, 0)),
    ],
    out_specs=pl.BlockSpec((_NO, _BM, _C), lambda i: (0, i, 0)),
    out_shape=jax.ShapeDtypeStruct((_NO, _NT, _C), jnp.float32),
)

_GP = 16768


def _lane_gather(v, i):
    return lax.gather(
        v, i[:, None],
        dimension_numbers=lax.GatherDimensionNumbers(
            offset_dims=(), collapsed_slice_dims=(0,), start_index_map=(0,)),
        slice_sizes=(1,),
        mode=lax.GatherScatterMode.PROMISE_IN_BOUNDS)
_DXY = [(dx, dy) for dx in (-1, 0, 1) for dy in (-1, 0, 1)]


@functools.lru_cache(maxsize=1)
def _get_sc_gather_sum():
    mesh = plsc.VectorSubcoreMesh(core_axis_name="c", subcore_axis_name="s")

    @functools.partial(
        pl.kernel,
        mesh=mesh,
        compiler_params=pltpu.CompilerParams(needs_layout_passes=False),
        out_type=jax.ShapeDtypeStruct((_N, _C), jnp.float32),
        scratch_types=[
            pltpu.VMEM((_NO, _PW), jnp.int32),
            pltpu.VMEM((_PW,), jnp.int32),
            pltpu.VMEM((_PW,), jnp.int32),
            pltpu.VMEM((_GP,), jnp.int32),
            pltpu.VMEM((_GP,), jnp.int32),
            pltpu.VMEM_SHARED((_NS, _GP), jnp.int32),
            pltpu.VMEM((_NO, _P, _C), jnp.float32),
            pltpu.VMEM((_NO, _P, _C), jnp.float32),
            pltpu.VMEM((_P, _C), jnp.float32),
            pltpu.VMEM((_P, _C), jnp.float32),
            pltpu.SemaphoreType.DMA,
            pltpu.SemaphoreType.DMA,
            pltpu.SemaphoreType.DMA,
            pltpu.SemaphoreType.DMA,
        ],
    )
    def _sc_gather_sum(z_hbm, xs_hbm, ys_hbm, out_hbm,
                       idx_v, xs_v, ys_v, map_v, tmp_v, shared_v, buf0, buf1,
                       acc0, acc1, sg0, sg1, sw0, sw1):
        sid = lax.axis_index("s")
        wid = lax.axis_index("c") * _NS + sid

        def fire(ch, buf, sem):
            for o in range(_NO):
                pltpu.async_copy(
                    z_hbm.at[idx_v.at[o, pl.ds(ch * _P, _P)]], buf.at[o], sem)

        def drain_gathers(buf, sem):
            for o in range(_NO):
                pltpu.make_async_copy(
                    z_hbm.at[pl.ds(0, _P)], buf.at[o], sem).wait()

        def accumulate(buf, acc):
            def row_body(r, c2):
                for c8 in range(_C // 16):
                    s = pl.ds(c8 * 16, 16)
                    v = buf[0, r, s]
                    for o in range(1, _NO):
                        v = v + buf[o, r, s]
                    acc[r, s] = v
                return c2

            lax.fori_loop(0, _P, row_body, 0)

        def process(ch, buf, acc, sg, sw):
            base = wid * _PW + ch * _P
            drain_gathers(buf, sg)

            @pl.when(ch >= 2)
            def _():
                pltpu.make_async_copy(
                    acc, out_hbm.at[pl.ds(base, _P)], sw).wait()

            accumulate(buf, acc)
            pltpu.async_copy(acc, out_hbm.at[pl.ds(base, _P)], sw)

            @pl.when(ch + 2 < _CH)
            def _():
                fire(ch + 2, buf, sg)

        pltpu.sync_copy(xs_hbm.at[pl.ds(wid * _PW, _PW)], xs_v)
        pltpu.sync_copy(ys_hbm.at[pl.ds(wid * _PW, _PW)], ys_v)

        def init_body(i, carry):
            map_v[pl.ds(i * 16, 16)] = jnp.full((16,), -1, jnp.int32)
            return carry

        lax.fori_loop(0, _GP // 16, init_body, 0)
        lane = lax.iota(jnp.int32, 16)

        def scat_body(g, carry):
            s = pl.ds(g * 16, 16)
            nf = xs_v[s] * _GY + ys_v[s]
            pidx = wid * _PW + g * 16 + lane
            ks, vs = plsc.sort_key_val(nf * 32768 + pidx, pidx)
            nfs = ks >> 15
            nxt = _lane_gather(nfs, jnp.minimum(lane + 1, 15))
            last = (nfs != nxt) | (lane == 15)
            plsc.store_scatter(map_v, [nfs], vs, mask=last)
            return carry

        lax.fori_loop(0, _PW // 16, scat_body, 0)

        pltpu.sync_copy(map_v, shared_v.at[sid])
        plsc.subcore_barrier()
        nb = _G // _PW
        sbase = (sid // nb) * nb
        for t in range(nb):

            @pl.when(sbase + t != sid)
            def _():
                pltpu.sync_copy(shared_v.at[sbase + t], tmp_v)

                def max_body(i, carry):
                    s = pl.ds(i * 16, 16)
                    map_v[s] = jnp.maximum(map_v[s], tmp_v[s])
                    return carry

                lax.fori_loop(0, _GP // 16, max_body, 0)

        def idx_body(g, carry):
            s = pl.ds(g * 16, 16)
            xv = xs_v[s]
            yv = ys_v[s]
            lane = lax.iota(jnp.int32, 16)
            pidx = wid * _PW + g * 16 + lane
            prow = _N + (pidx & (_BM - 1))
            for o, (dx, dy) in enumerate(_DXY):
                nx = xv + dx
                ny = yv + dy
                valid = (nx >= 0) & (nx < _GX) & (ny >= 0) & (ny < _GY)
                nf = (jnp.clip(nx, 0, _GX - 1) * _GY
                      + jnp.clip(ny, 0, _GY - 1))
                j = plsc.load_gather(map_v, [nf])
                valid = valid & (j >= 0)
                idx_v[o, s] = jnp.where(valid, o * _NT + j, prow)
            return carry

        lax.fori_loop(0, _PW // 16, idx_body, 0)
        fire(0, buf0, sg0)
        fire(1, buf1, sg1)

        def pair_body(k, carry):
            process(2 * k, buf0, acc0, sg0, sw0)
            process(2 * k + 1, buf1, acc1, sg1, sw1)
            return carry

        lax.fori_loop(0, _CH // 2, pair_body, 0)
        pltpu.make_async_copy(
            acc0, out_hbm.at[pl.ds(wid * _PW, _P)], sw0).wait()
        pltpu.make_async_copy(
            acc1, out_hbm.at[pl.ds(wid * _PW, _P)], sw1).wait()

    return _sc_gather_sum


def kernel(instance_feature, anchor, W):
    b, g = instance_feature.shape[:2]
    anchor_xy = jax.nn.sigmoid(jnp.clip(anchor[..., :2], -10.0, 10.0)).reshape(-1, 2)
    grid_size = 1.0 / jnp.asarray(_FM, dtype=jnp.float32)
    indices = ((anchor_xy - anchor_xy.min(axis=0, keepdims=True)) / grid_size
               ).astype(jnp.int32)
    feats = instance_feature.reshape(b * g, -1).astype(jnp.float32)

    xs = indices[:, 0]
    ys = indices[:, 1]

    feats_p = jnp.concatenate(
        [feats, jnp.zeros((_NT - _N, _C), jnp.float32)], axis=0
    ).astype(_mm_in_dtype)
    w2 = W.reshape(_NO, _C, _C).astype(_mm_in_dtype)

    z = _mm(feats_p, w2).reshape(_NO * _NT, _C)
    out = _get_sc_gather_sum()(z, xs, ys)
    return out.reshape(b, g, -1)


# speedup vs baseline: 1.1143x; 1.1143x over previous
import functools

import jax
import jax.numpy as jnp
from jax import lax
from jax.experimental import pallas as pl
from jax.experimental.pallas import tpu as pltpu
from jax.experimental.pallas import tpu_sc as plsc

_B, _G, _C, _K = 4, 8192, 128, 3
_FM = (128, 128)
_GX, _GY = _FM[0] + 1, _FM[1] + 1
_N = _B * _G
_BM = 512
_NT = _N + _BM
_NO = _K * _K
_NC, _NS = 2, 16
_NW = _NC * _NS
_PW = _N // _NW
_P = 16
_CH = _PW // _P


_mm_in_dtype = jnp.bfloat16


def _mm_body(f_ref, w_ref, z_ref):
    f = f_ref[...]
    for o in range(_NO):
        z_ref[o] = jnp.dot(f, w_ref[o], preferred_element_type=jnp.float32)


_mm = pl.pallas_call(
    _mm_body,
    grid=(_NT // _BM,),
    in_specs=[
        pl.BlockSpec((_BM, _C), lambda i: (i, 0)),
        pl.BlockSpec((_NO, _C, _C), lambda i: (0, 0, 0)),
    ],
    out_specs=pl.BlockSpec((_NO, _BM, _C), lambda i: (0, i, 0)),
    out_shape=jax.ShapeDtypeStruct((_NO, _NT, _C), jnp.float32),
)

_GP = 16768


def _lane_gather(v, i):
    return lax.gather(
        v, i[:, None],
        dimension_numbers=lax.GatherDimensionNumbers(
            offset_dims=(), collapsed_slice_dims=(0,), start_index_map=(0,)),
        slice_sizes=(1,),
        mode=lax.GatherScatterMode.PROMISE_IN_BOUNDS)
_DXY = [(dx, dy) for dx in (-1, 0, 1) for dy in (-1, 0, 1)]


@functools.lru_cache(maxsize=1)
def _get_sc_gather_sum():
    mesh = plsc.VectorSubcoreMesh(core_axis_name="c", subcore_axis_name="s")

    @functools.partial(
        pl.kernel,
        mesh=mesh,
        compiler_params=pltpu.CompilerParams(needs_layout_passes=False),
        out_type=jax.ShapeDtypeStruct((_N, _C), jnp.float32),
        scratch_types=[
            pltpu.VMEM((_NO, _PW), jnp.int32),
            pltpu.VMEM((_PW,), jnp.int32),
            pltpu.VMEM((_PW,), jnp.int32),
            pltpu.VMEM((_GP,), jnp.int32),
            pltpu.VMEM((_GP,), jnp.int32),
            pltpu.VMEM_SHARED((_NS, _GP), jnp.int32),
            pltpu.VMEM((_NO, _P, _C), jnp.float32),
            pltpu.VMEM((_NO, _P, _C), jnp.float32),
            pltpu.VMEM((_P, _C), jnp.float32),
            pltpu.VMEM((_P, _C), jnp.float32),
            pltpu.SemaphoreType.DMA,
            pltpu.SemaphoreType.DMA,
            pltpu.SemaphoreType.DMA,
            pltpu.SemaphoreType.DMA,
        ],
    )
    def _sc_gather_sum(z_hbm, xs_hbm, ys_hbm, out_hbm,
                       idx_v, xs_v, ys_v, map_v, tmp_v, shared_v, buf0, buf1,
                       acc0, acc1, sg0, sg1, sw0, sw1):
        sid = lax.axis_index("s")
        wid = lax.axis_index("c") * _NS + sid

        def fire(ch, buf, sem):
            for o in range(_NO):
                pltpu.async_copy(
                    z_hbm.at[idx_v.at[o, pl.ds(ch * _P, _P)]], buf.at[o], sem)

        def drain_gathers(buf, sem):
            for o in range(_NO):
                pltpu.make_async_copy(
                    z_hbm.at[pl.ds(0, _P)], buf.at[o], sem).wait()

        def accumulate(buf, acc):
            def row_body(r, c2):
                for c8 in range(_C // 16):
                    s = pl.ds(c8 * 16, 16)
                    v = buf[0, r, s]
                    for o in range(1, _NO):
                        v = v + buf[o, r, s]
                    acc[r, s] = v
                return c2

            lax.fori_loop(0, _P, row_body, 0)

        def process(ch, buf, acc, sg, sw):
            base = wid * _PW + ch * _P
            drain_gathers(buf, sg)

            @pl.when(ch >= 2)
            def _():
                pltpu.make_async_copy(
                    acc, out_hbm.at[pl.ds(base, _P)], sw).wait()

            accumulate(buf, acc)
            pltpu.async_copy(acc, out_hbm.at[pl.ds(base, _P)], sw)

            @pl.when(ch + 2 < _CH)
            def _():
                fire(ch + 2, buf, sg)

        pltpu.sync_copy(xs_hbm.at[pl.ds(wid * _PW, _PW)], xs_v)
        pltpu.sync_copy(ys_hbm.at[pl.ds(wid * _PW, _PW)], ys_v)

        def init_body(i, carry):
            neg1 = jnp.full((16,), -1, jnp.int32)
            for u in range(8):
                map_v[pl.ds(i * 128 + u * 16, 16)] = neg1
            return carry

        lax.fori_loop(0, _GP // 128, init_body, 0)
        lane = lax.iota(jnp.int32, 16)

        def scat_body(g, carry):
            s = pl.ds(g * 16, 16)
            nf = xs_v[s] * _GY + ys_v[s]
            pidx = wid * _PW + g * 16 + lane
            ks, vs = plsc.sort_key_val(nf * 32768 + pidx, pidx)
            nfs = ks >> 15
            nxt = _lane_gather(nfs, jnp.minimum(lane + 1, 15))
            last = (nfs != nxt) | (lane == 15)
            plsc.store_scatter(map_v, [nfs], vs, mask=last)
            return carry

        lax.fori_loop(0, _PW // 16, scat_body, 0)

        pltpu.sync_copy(map_v, shared_v.at[sid])
        plsc.subcore_barrier()
        nb = _G // _PW
        sbase = (sid // nb) * nb
        for t in range(nb):

            @pl.when(sbase + t != sid)
            def _():
                pltpu.sync_copy(shared_v.at[sbase + t], tmp_v)

                def max_body(i, carry):
                    for u in range(8):
                        s = pl.ds(i * 128 + u * 16, 16)
                        map_v[s] = jnp.maximum(map_v[s], tmp_v[s])
                    return carry

                lax.fori_loop(0, _GP // 128, max_body, 0)

        def idx_body(g, carry):
            s = pl.ds(g * 16, 16)
            xv = xs_v[s]
            yv = ys_v[s]
            lane = lax.iota(jnp.int32, 16)
            pidx = wid * _PW + g * 16 + lane
            prow = _N + (pidx & (_BM - 1))
            for o, (dx, dy) in enumerate(_DXY):
                nx = xv + dx
                ny = yv + dy
                valid = (nx >= 0) & (nx < _GX) & (ny >= 0) & (ny < _GY)
                nf = (jnp.clip(nx, 0, _GX - 1) * _GY
                      + jnp.clip(ny, 0, _GY - 1))
                j = plsc.load_gather(map_v, [nf])
                valid = valid & (j >= 0)
                idx_v[o, s] = jnp.where(valid, o * _NT + j, prow)
            return carry

        lax.fori_loop(0, _PW // 16, idx_body, 0)
        fire(0, buf0, sg0)
        fire(1, buf1, sg1)

        def pair_body(k, carry):
            process(2 * k, buf0, acc0, sg0, sw0)
            process(2 * k + 1, buf1, acc1, sg1, sw1)
            return carry

        lax.fori_loop(0, _CH // 2, pair_body, 0)
        pltpu.make_async_copy(
            acc0, out_hbm.at[pl.ds(wid * _PW, _P)], sw0).wait()
        pltpu.make_async_copy(
            acc1, out_hbm.at[pl.ds(wid * _PW, _P)], sw1).wait()

    return _sc_gather_sum


def kernel(instance_feature, anchor, W):
    b, g = instance_feature.shape[:2]
    anchor_xy = jax.nn.sigmoid(jnp.clip(anchor[..., :2], -10.0, 10.0)).reshape(-1, 2)
    grid_size = 1.0 / jnp.asarray(_FM, dtype=jnp.float32)
    indices = ((anchor_xy - anchor_xy.min(axis=0, keepdims=True)) / grid_size
               ).astype(jnp.int32)
    feats = instance_feature.reshape(b * g, -1).astype(jnp.float32)

    xs = indices[:, 0]
    ys = indices[:, 1]

    feats_p = jnp.concatenate(
        [feats, jnp.zeros((_NT - _N, _C), jnp.float32)], axis=0
    ).astype(_mm_in_dtype)
    w2 = W.reshape(_NO, _C, _C).astype(_mm_in_dtype)

    z = _mm(feats_p, w2).reshape(_NO * _NT, _C)
    out = _get_sc_gather_sum()(z, xs, ys)
    return out.reshape(b, g, -1)


# speedup vs baseline: 1.2420x; 1.1145x over previous
import functools

import jax
import jax.numpy as jnp
from jax import lax
from jax.experimental import pallas as pl
from jax.experimental.pallas import tpu as pltpu
from jax.experimental.pallas import tpu_sc as plsc

_B, _G, _C, _K = 4, 8192, 128, 3
_FM = (128, 128)
_GX, _GY = _FM[0] + 1, _FM[1] + 1
_N = _B * _G
_BM = 512
_NT = _N + _BM
_NO = _K * _K
_NC, _NS = 2, 16
_NW = _NC * _NS
_PW = _N // _NW
_P = 32
_CH = _PW // _P


_mm_in_dtype = jnp.bfloat16


def _mm_body(f_ref, w_ref, z_ref):
    f = f_ref[...]
    for o in range(_NO):
        z_ref[o] = jnp.dot(f, w_ref[o], preferred_element_type=jnp.float32)


_mm = pl.pallas_call(
    _mm_body,
    grid=(_NT // _BM,),
    in_specs=[
        pl.BlockSpec((_BM, _C), lambda i: (i, 0)),
        pl.BlockSpec((_NO, _C, _C), lambda i: (0, 0, 0)),
    ],
    out_specs=pl.BlockSpec((_NO, _BM, _C), lambda i: (0, i, 0)),
    out_shape=jax.ShapeDtypeStruct((_NO, _NT, _C), jnp.float32),
)

_GP = 16768


def _lane_gather(v, i):
    return lax.gather(
        v, i[:, None],
        dimension_numbers=lax.GatherDimensionNumbers(
            offset_dims=(), collapsed_slice_dims=(0,), start_index_map=(0,)),
        slice_sizes=(1,),
        mode=lax.GatherScatterMode.PROMISE_IN_BOUNDS)
_DXY = [(dx, dy) for dx in (-1, 0, 1) for dy in (-1, 0, 1)]


@functools.lru_cache(maxsize=1)
def _get_sc_build_idx():
    mesh = plsc.VectorSubcoreMesh(core_axis_name="c", subcore_axis_name="s")

    @functools.partial(
        pl.kernel,
        mesh=mesh,
        compiler_params=pltpu.CompilerParams(needs_layout_passes=False),
        out_type=jax.ShapeDtypeStruct((_NW, _NO, _PW), jnp.int32),
        scratch_types=[
            pltpu.VMEM((_NO, _PW), jnp.int32),
            pltpu.VMEM((_PW,), jnp.int32),
            pltpu.VMEM((_PW,), jnp.int32),
            pltpu.VMEM((_GP,), jnp.int32),
            pltpu.VMEM((_GP,), jnp.int32),
            pltpu.VMEM_SHARED((_NS, _GP), jnp.int32),
        ],
    )
    def _sc_build_idx(xs_hbm, ys_hbm, gidx_hbm,
                      idx_v, xs_v, ys_v, map_v, tmp_v, shared_v):
        sid = lax.axis_index("s")
        wid = lax.axis_index("c") * _NS + sid

        pltpu.sync_copy(xs_hbm.at[pl.ds(wid * _PW, _PW)], xs_v)
        pltpu.sync_copy(ys_hbm.at[pl.ds(wid * _PW, _PW)], ys_v)

        def init_body(i, carry):
            neg1 = jnp.full((16,), -1, jnp.int32)
            for u in range(8):
                map_v[pl.ds(i * 128 + u * 16, 16)] = neg1
            return carry

        lax.fori_loop(0, _GP // 128, init_body, 0)
        lane = lax.iota(jnp.int32, 16)

        def scat_body(g, carry):
            s = pl.ds(g * 16, 16)
            nf = xs_v[s] * _GY + ys_v[s]
            pidx = wid * _PW + g * 16 + lane
            ks, vs = plsc.sort_key_val(nf * 32768 + pidx, pidx)
            nfs = ks >> 15
            nxt = _lane_gather(nfs, jnp.minimum(lane + 1, 15))
            last = (nfs != nxt) | (lane == 15)
            plsc.store_scatter(map_v, [nfs], vs, mask=last)
            return carry

        lax.fori_loop(0, _PW // 16, scat_body, 0)

        pltpu.sync_copy(map_v, shared_v.at[sid])
        plsc.subcore_barrier()
        nb = _G // _PW
        sbase = (sid // nb) * nb
        for t in range(nb):

            @pl.when(sbase + t != sid)
            def _():
                pltpu.sync_copy(shared_v.at[sbase + t], tmp_v)

                def max_body(i, carry):
                    for u in range(8):
                        s = pl.ds(i * 128 + u * 16, 16)
                        map_v[s] = jnp.maximum(map_v[s], tmp_v[s])
                    return carry

                lax.fori_loop(0, _GP // 128, max_body, 0)

        def idx_body(g, carry):
            s = pl.ds(g * 16, 16)
            xv = xs_v[s]
            yv = ys_v[s]
            lane = lax.iota(jnp.int32, 16)
            pidx = wid * _PW + g * 16 + lane
            prow = _N + (pidx & (_BM - 1))
            for o, (dx, dy) in enumerate(_DXY):
                nx = xv + dx
                ny = yv + dy
                valid = (nx >= 0) & (nx < _GX) & (ny >= 0) & (ny < _GY)
                nf = (jnp.clip(nx, 0, _GX - 1) * _GY
                      + jnp.clip(ny, 0, _GY - 1))
                j = plsc.load_gather(map_v, [nf])
                valid = valid & (j >= 0)
                idx_v[o, s] = jnp.where(valid, o * _NT + j, prow)
            return carry

        lax.fori_loop(0, _PW // 16, idx_body, 0)
        pltpu.sync_copy(idx_v, gidx_hbm.at[wid])

    return _sc_build_idx


@functools.lru_cache(maxsize=1)
def _get_sc_gather_sum():
    mesh = plsc.VectorSubcoreMesh(core_axis_name="c", subcore_axis_name="s")

    @functools.partial(
        pl.kernel,
        mesh=mesh,
        compiler_params=pltpu.CompilerParams(needs_layout_passes=False),
        out_type=jax.ShapeDtypeStruct((_N, _C), jnp.float32),
        scratch_types=[
            pltpu.VMEM((_NO, _PW), jnp.int32),
            pltpu.VMEM((_NO, _P, _C), jnp.float32),
            pltpu.VMEM((_NO, _P, _C), jnp.float32),
            pltpu.VMEM((_P, _C), jnp.float32),
            pltpu.VMEM((_P, _C), jnp.float32),
            pltpu.SemaphoreType.DMA,
            pltpu.SemaphoreType.DMA,
            pltpu.SemaphoreType.DMA,
            pltpu.SemaphoreType.DMA,
        ],
    )
    def _sc_gather_sum(z_hbm, gidx_hbm, out_hbm,
                       idx_v, buf0, buf1, acc0, acc1, sg0, sg1, sw0, sw1):
        wid = lax.axis_index("c") * _NS + lax.axis_index("s")

        def fire(ch, buf, sem):
            for o in range(_NO):
                pltpu.async_copy(
                    z_hbm.at[idx_v.at[o, pl.ds(ch * _P, _P)]], buf.at[o], sem)

        def drain_gathers(buf, sem):
            for o in range(_NO):
                pltpu.make_async_copy(
                    z_hbm.at[pl.ds(0, _P)], buf.at[o], sem).wait()

        def accumulate(buf, acc):
            def row_body(r, c2):
                for c8 in range(_C // 16):
                    s = pl.ds(c8 * 16, 16)
                    v = buf[0, r, s]
                    for o in range(1, _NO):
                        v = v + buf[o, r, s]
                    acc[r, s] = v
                return c2

            lax.fori_loop(0, _P, row_body, 0)

        def process(ch, buf, acc, sg, sw):
            base = wid * _PW + ch * _P
            drain_gathers(buf, sg)

            @pl.when(ch >= 2)
            def _():
                pltpu.make_async_copy(
                    acc, out_hbm.at[pl.ds(base, _P)], sw).wait()

            accumulate(buf, acc)
            pltpu.async_copy(acc, out_hbm.at[pl.ds(base, _P)], sw)

            @pl.when(ch + 2 < _CH)
            def _():
                fire(ch + 2, buf, sg)

        pltpu.sync_copy(gidx_hbm.at[wid], idx_v)
        fire(0, buf0, sg0)
        fire(1, buf1, sg1)

        def pair_body(k, carry):
            process(2 * k, buf0, acc0, sg0, sw0)
            process(2 * k + 1, buf1, acc1, sg1, sw1)
            return carry

        lax.fori_loop(0, _CH // 2, pair_body, 0)
        pltpu.make_async_copy(
            acc0, out_hbm.at[pl.ds(wid * _PW, _P)], sw0).wait()
        pltpu.make_async_copy(
            acc1, out_hbm.at[pl.ds(wid * _PW, _P)], sw1).wait()

    return _sc_gather_sum


def kernel(instance_feature, anchor, W):
    b, g = instance_feature.shape[:2]
    anchor_xy = jax.nn.sigmoid(jnp.clip(anchor[..., :2], -10.0, 10.0)).reshape(-1, 2)
    grid_size = 1.0 / jnp.asarray(_FM, dtype=jnp.float32)
    indices = ((anchor_xy - anchor_xy.min(axis=0, keepdims=True)) / grid_size
               ).astype(jnp.int32)
    feats = instance_feature.reshape(b * g, -1).astype(jnp.float32)

    xs = indices[:, 0]
    ys = indices[:, 1]

    feats_p = jnp.concatenate(
        [feats, jnp.zeros((_NT - _N, _C), jnp.float32)], axis=0
    ).astype(_mm_in_dtype)
    w2 = W.reshape(_NO, _C, _C).astype(_mm_in_dtype)

    gidx = _get_sc_build_idx()(xs, ys)
    z = _mm(feats_p, w2).reshape(_NO * _NT, _C)
    out = _get_sc_gather_sum()(z, gidx)
    return out.reshape(b, g, -1)


# speedup vs baseline: 1.2486x; 1.0053x over previous
import functools

import jax
import jax.numpy as jnp
from jax import lax
from jax.experimental import pallas as pl
from jax.experimental.pallas import tpu as pltpu
from jax.experimental.pallas import tpu_sc as plsc

_B, _G, _C, _K = 4, 8192, 128, 3
_FM = (128, 128)
_GX, _GY = _FM[0] + 1, _FM[1] + 1
_N = _B * _G
_BM = 512
_NT = _N + _BM
_NO = _K * _K
_NC, _NS = 2, 16
_NW = _NC * _NS
_PW = _N // _NW
_P = 32
_CH = _PW // _P


_mm_in_dtype = jnp.bfloat16


def _mm_body(f_ref, w_ref, z_ref):
    f = f_ref[...]
    for o in range(_NO):
        z_ref[o] = jnp.dot(f, w_ref[o], preferred_element_type=jnp.float32)


_mm = pl.pallas_call(
    _mm_body,
    grid=(_NT // _BM,),
    in_specs=[
        pl.BlockSpec((_BM, _C), lambda i: (i, 0)),
        pl.BlockSpec((_NO, _C, _C), lambda i: (0, 0, 0)),
    ],
    out_specs=pl.BlockSpec((_NO, _BM, _C), lambda i: (0, i, 0)),
    out_shape=jax.ShapeDtypeStruct((_NO, _NT, _C), jnp.float32),
)

_GP = 16768


def _lane_gather(v, i):
    return lax.gather(
        v, i[:, None],
        dimension_numbers=lax.GatherDimensionNumbers(
            offset_dims=(), collapsed_slice_dims=(0,), start_index_map=(0,)),
        slice_sizes=(1,),
        mode=lax.GatherScatterMode.PROMISE_IN_BOUNDS)
_DXY = [(dx, dy) for dx in (-1, 0, 1) for dy in (-1, 0, 1)]


@functools.lru_cache(maxsize=1)
def _get_sc_build_idx():
    mesh = plsc.VectorSubcoreMesh(core_axis_name="c", subcore_axis_name="s")

    @functools.partial(
        pl.kernel,
        mesh=mesh,
        compiler_params=pltpu.CompilerParams(needs_layout_passes=False),
        out_type=jax.ShapeDtypeStruct((_NW, _NO, _PW), jnp.int32),
        scratch_types=[
            pltpu.VMEM((_NO, _PW), jnp.int32),
            pltpu.VMEM((_PW,), jnp.int32),
            pltpu.VMEM((_PW,), jnp.int32),
            pltpu.VMEM((_GP,), jnp.int32),
            pltpu.VMEM((_GP,), jnp.int32),
            pltpu.VMEM_SHARED((_NS, _GP), jnp.int32),
        ],
    )
    def _sc_build_idx(xs_hbm, ys_hbm, gidx_hbm,
                      idx_v, xs_v, ys_v, map_v, tmp_v, shared_v):
        sid = lax.axis_index("s")
        wid = lax.axis_index("c") * _NS + sid

        pltpu.sync_copy(xs_hbm.at[pl.ds(wid * _PW, _PW)], xs_v)
        pltpu.sync_copy(ys_hbm.at[pl.ds(wid * _PW, _PW)], ys_v)

        def init_body(i, carry):
            neg1 = jnp.full((16,), -1, jnp.int32)
            for u in range(8):
                map_v[pl.ds(i * 128 + u * 16, 16)] = neg1
            return carry

        lax.fori_loop(0, _GP // 128, init_body, 0)
        lane = lax.iota(jnp.int32, 16)

        def scat_body(g, carry):
            s = pl.ds(g * 16, 16)
            nf = xs_v[s] * _GY + ys_v[s]
            pidx = wid * _PW + g * 16 + lane
            ks, vs = plsc.sort_key_val(nf * 32768 + pidx, pidx)
            nfs = ks >> 15
            nxt = _lane_gather(nfs, jnp.minimum(lane + 1, 15))
            last = (nfs != nxt) | (lane == 15)
            plsc.store_scatter(map_v, [nfs], vs, mask=last)
            return carry

        lax.fori_loop(0, _PW // 16, scat_body, 0)

        pltpu.sync_copy(map_v, shared_v.at[sid])
        plsc.subcore_barrier()
        nb = _G // _PW
        sbase = (sid // nb) * nb
        for t in range(nb):

            @pl.when(sbase + t != sid)
            def _():
                pltpu.sync_copy(shared_v.at[sbase + t], tmp_v)

                def max_body(i, carry):
                    for u in range(8):
                        s = pl.ds(i * 128 + u * 16, 16)
                        map_v[s] = jnp.maximum(map_v[s], tmp_v[s])
                    return carry

                lax.fori_loop(0, _GP // 128, max_body, 0)

        def idx_body(g, carry):
            s = pl.ds(g * 16, 16)
            xv = xs_v[s]
            yv = ys_v[s]
            lane = lax.iota(jnp.int32, 16)
            pidx = wid * _PW + g * 16 + lane
            prow = _N + (pidx & (_BM - 1))
            for o, (dx, dy) in enumerate(_DXY):
                nx = xv + dx
                ny = yv + dy
                valid = (nx >= 0) & (nx < _GX) & (ny >= 0) & (ny < _GY)
                nf = (jnp.clip(nx, 0, _GX - 1) * _GY
                      + jnp.clip(ny, 0, _GY - 1))
                j = plsc.load_gather(map_v, [nf])
                valid = valid & (j >= 0)
                idx_v[o, s] = jnp.where(valid, o * _NT + j, prow)
            return carry

        lax.fori_loop(0, _PW // 16, idx_body, 0)
        pltpu.sync_copy(idx_v, gidx_hbm.at[wid])

    return _sc_build_idx


@functools.lru_cache(maxsize=1)
def _get_sc_gather_sum():
    mesh = plsc.VectorSubcoreMesh(core_axis_name="c", subcore_axis_name="s")

    @functools.partial(
        pl.kernel,
        mesh=mesh,
        compiler_params=pltpu.CompilerParams(needs_layout_passes=False),
        out_type=jax.ShapeDtypeStruct((_N, _C), jnp.float32),
        scratch_types=[
            pltpu.VMEM((_NO, _PW), jnp.int32),
            pltpu.VMEM((_NO, _P, _C), jnp.float32),
            pltpu.VMEM((_NO, _P, _C), jnp.float32),
            pltpu.VMEM((_P, _C), jnp.float32),
            pltpu.VMEM((_P, _C), jnp.float32),
            pltpu.SemaphoreType.DMA,
            pltpu.SemaphoreType.DMA,
            pltpu.SemaphoreType.DMA,
            pltpu.SemaphoreType.DMA,
        ],
    )
    def _sc_gather_sum(z_hbm, gidx_hbm, out_hbm,
                       idx_v, buf0, buf1, acc0, acc1, sg0, sg1, sw0, sw1):
        wid = lax.axis_index("c") * _NS + lax.axis_index("s")

        def fire(ch, buf, sem):
            for o in range(_NO):
                pltpu.async_copy(
                    z_hbm.at[idx_v.at[o, pl.ds(ch * _P, _P)]], buf.at[o], sem)

        def drain_gathers(buf, sem):
            for o in range(_NO):
                pltpu.make_async_copy(
                    z_hbm.at[pl.ds(0, _P)], buf.at[o], sem).wait()

        def accumulate(buf, acc):
            @plsc.parallel_loop(0, _P, unroll=2)
            def _(r):
                for c8 in range(_C // 16):
                    s = pl.ds(c8 * 16, 16)
                    v = buf[0, r, s]
                    for o in range(1, _NO):
                        v = v + buf[o, r, s]
                    acc[r, s] = v

        def process(ch, buf, acc, sg, sw):
            base = wid * _PW + ch * _P
            drain_gathers(buf, sg)

            @pl.when(ch >= 2)
            def _():
                pltpu.make_async_copy(
                    acc, out_hbm.at[pl.ds(base, _P)], sw).wait()

            accumulate(buf, acc)
            pltpu.async_copy(acc, out_hbm.at[pl.ds(base, _P)], sw)

            @pl.when(ch + 2 < _CH)
            def _():
                fire(ch + 2, buf, sg)

        pltpu.sync_copy(gidx_hbm.at[wid], idx_v)
        fire(0, buf0, sg0)
        fire(1, buf1, sg1)

        def pair_body(k, carry):
            process(2 * k, buf0, acc0, sg0, sw0)
            process(2 * k + 1, buf1, acc1, sg1, sw1)
            return carry

        lax.fori_loop(0, _CH // 2, pair_body, 0)
        pltpu.make_async_copy(
            acc0, out_hbm.at[pl.ds(wid * _PW, _P)], sw0).wait()
        pltpu.make_async_copy(
            acc1, out_hbm.at[pl.ds(wid * _PW, _P)], sw1).wait()

    return _sc_gather_sum


def kernel(instance_feature, anchor, W):
    b, g = instance_feature.shape[:2]
    anchor_xy = jax.nn.sigmoid(jnp.clip(anchor[..., :2], -10.0, 10.0)).reshape(-1, 2)
    grid_size = 1.0 / jnp.asarray(_FM, dtype=jnp.float32)
    indices = ((anchor_xy - anchor_xy.min(axis=0, keepdims=True)) / grid_size
               ).astype(jnp.int32)
    feats = instance_feature.reshape(b * g, -1).astype(jnp.float32)

    xs = indices[:, 0]
    ys = indices[:, 1]

    feats_p = jnp.concatenate(
        [feats, jnp.zeros((_NT - _N, _C), jnp.float32)], axis=0
    ).astype(_mm_in_dtype)
    w2 = W.reshape(_NO, _C, _C).astype(_mm_in_dtype)

    gidx = _get_sc_build_idx()(xs, ys)
    z = _mm(feats_p, w2).reshape(_NO * _NT, _C)
    out = _get_sc_gather_sum()(z, gidx)
    return out.reshape(b, g, -1)
